# baseline (speedup 1.0000x reference)
import functools
import jax
import jax.numpy as jnp
from jax import lax
from jax.experimental import pallas as pl
from jax.experimental.pallas import tpu as pltpu
from jax.experimental.pallas import tpu_sc as plsc

N0 = 50000
N1 = 20000
NNZ = 800000
H = 64
NTILES = 16
NCORES = 2
NW = NCORES * NTILES
SHIFT = 15
MASK = (1 << SHIFT) - 1

_CHUNK = 40000
_NCH = NNZ // _CHUNK
_UNR = 10


def _make_seg(n_src, n_dst, gather_hi):
    mesh = plsc.VectorSubcoreMesh(core_axis_name="c", subcore_axis_name="s")

    @functools.partial(
        pl.kernel,
        out_type=jax.ShapeDtypeStruct((H * n_dst,), jnp.float32),
        mesh=mesh,
        compiler_params=pltpu.CompilerParams(needs_layout_passes=False),
        scratch_types=[
            pltpu.VMEM((n_src,), jnp.float32),
            pltpu.VMEM((n_dst,), jnp.float32),
            pltpu.VMEM((_CHUNK,), jnp.int32),
        ],
    )
    def seg(tab_hbm, pairs_hbm, out_hbm, tab_v, acc_v, idx_v):
        c = lax.axis_index("c")
        s = lax.axis_index("s")
        wid = c * NTILES + s

        for sweep in range(2):
            f = sweep * NW + wid
            pltpu.sync_copy(tab_hbm.at[pl.ds(pl.multiple_of(f * n_src, 8),
                                             n_src)], tab_v)

            def zero_body(i, _):
                acc_v[pl.ds(pl.multiple_of(i * 16, 8), 16)] = (
                    jnp.zeros((16,), jnp.float32))
                return 0
            lax.fori_loop(0, n_dst // 16, zero_body, 0)

            def chunk_body(a, _):
                pltpu.sync_copy(
                    pairs_hbm.at[pl.ds(pl.multiple_of(a * _CHUNK, 8),
                                       _CHUNK)], idx_v)

                def grp_body(g, _):
                    for u in range(_UNR):
                        base = (g * _UNR + u) * 16
                        p = idx_v[pl.ds(pl.multiple_of(base, 8), 16)]
                        hi = lax.shift_right_logical(p, SHIFT)
                        lo = lax.bitwise_and(p, MASK)
                        src = hi if gather_hi else lo
                        dst = lo if gather_hi else hi
                        vals = plsc.load_gather(tab_v, [src])
                        plsc.addupdate_scatter(acc_v, [dst], vals)
                    return 0
                lax.fori_loop(0, _CHUNK // (16 * _UNR), grp_body, 0)
                return 0
            lax.fori_loop(0, _NCH, chunk_body, 0)

            pltpu.sync_copy(acc_v,
                            out_hbm.at[pl.ds(pl.multiple_of(f * n_dst, 8),
                                             n_dst)])

    return seg


_seg_v2e = _make_seg(N0, N1, True)
_seg_e2v = _make_seg(N1, N0, False)


_DPT = NNZ // NW
_DG = _DPT // 16


def _make_deg():
    mesh = plsc.VectorSubcoreMesh(core_axis_name="c", subcore_axis_name="s")

    @functools.partial(
        pl.kernel,
        out_type=jax.ShapeDtypeStruct((NW * N0,), jnp.float32),
        mesh=mesh,
        compiler_params=pltpu.CompilerParams(needs_layout_passes=False),
        scratch_types=[
            pltpu.VMEM((N0,), jnp.float32),
            pltpu.VMEM((_DPT + 16,), jnp.int32),
        ],
    )
    def deg(pairs_hbm, out_hbm, acc_v, idx_v):
        c = lax.axis_index("c")
        s = lax.axis_index("s")
        wid = c * NTILES + s

        def zero_body(i, _):
            acc_v[pl.ds(pl.multiple_of(i * 16, 8), 16)] = (
                jnp.zeros((16,), jnp.float32))
            return 0
        lax.fori_loop(0, N0 // 16, zero_body, 0)

        pltpu.sync_copy(
            pairs_hbm.at[pl.ds(pl.multiple_of(wid * _DPT, 8), _DPT)],
            idx_v.at[pl.ds(0, _DPT)])

        ones = jnp.ones((16,), jnp.float32)

        def grp_body(g, _):
            p = idx_v[pl.ds(pl.multiple_of(g * 16, 8), 16)]
            row = lax.shift_right_logical(p, SHIFT)
            plsc.addupdate_scatter(acc_v, [row], ones)
            return 0
        lax.fori_loop(0, _DG, grp_body, 0)

        p = idx_v[pl.ds(pl.multiple_of(_DG * 16, 8), 16)]
        row = lax.shift_right_logical(p, SHIFT)
        msk = lax.iota(jnp.int32, 16) < (_DPT - _DG * 16)
        plsc.addupdate_scatter(acc_v, [row], ones, mask=msk)

        pltpu.sync_copy(acc_v,
                        out_hbm.at[pl.ds(pl.multiple_of(wid * N0, 8), N0)])

    return deg


_deg_kernel = _make_deg()


_RB = 1024
_NB0 = (N0 + _RB - 1) // _RB
_NB1 = (N1 + _RB - 1) // _RB
_V0 = N0 - (_NB0 - 1) * _RB
_V1 = N1 - (_NB1 - 1) * _RB


def _pack_body(r_ref, c_ref, o_ref):
    o_ref[...] = lax.shift_left(r_ref[...], SHIFT) | c_ref[...]


def _pack_pairs(rows, cols):
    r2 = rows.reshape(NNZ // 128, 128)
    c2 = cols.reshape(NNZ // 128, 128)
    out = pl.pallas_call(
        _pack_body,
        out_shape=jax.ShapeDtypeStruct((NNZ // 128, 128), jnp.int32),
    )(r2, c2)
    return out.reshape(NNZ)


def _tcd_body(dp_ref, o_ref):
    d = jnp.sum(dp_ref[...], axis=0, keepdims=True)
    o_ref[...] = jnp.maximum(d, 1.0)


def _tcd(deg_partials):
    return pl.pallas_call(
        _tcd_body,
        grid=(_NB0,),
        in_specs=[pl.BlockSpec((NW, _RB), lambda i: (0, i))],
        out_specs=pl.BlockSpec((1, _RB), lambda i: (0, i)),
        out_shape=jax.ShapeDtypeStruct((1, N0), jnp.float32),
    )(deg_partials)


def _tc0_body(x_ref, wi_ref, bi_ref, wl_ref, blc_ref, o_ref):
    t = jnp.dot(x_ref[...], wi_ref[...],
                preferred_element_type=jnp.float32) + bi_ref[...]
    yt = lax.dot_general(wl_ref[...], t, (((0,), (1,)), ((), ())),
                         preferred_element_type=jnp.float32)
    o_ref[...] = yt + blc_ref[...]


def _tc0(x0, W_in0, b_in0, Wl0, bl0c):
    return pl.pallas_call(
        _tc0_body,
        grid=(_NB0,),
        in_specs=[
            pl.BlockSpec((_RB, 32), lambda i: (i, 0)),
            pl.BlockSpec((32, H), lambda i: (0, 0)),
            pl.BlockSpec((1, H), lambda i: (0, 0)),
            pl.BlockSpec((H, H), lambda i: (0, 0)),
            pl.BlockSpec((H, 1), lambda i: (0, 0)),
        ],
        out_specs=pl.BlockSpec((H, _RB), lambda i: (0, i)),
        out_shape=jax.ShapeDtypeStruct((H, N0), jnp.float32),
    )(x0, W_in0, b_in0, Wl0, bl0c)


def _tc1_body(xp_ref, msg_ref, dinv_ref, wl_ref, blc_ref, o_ref):
    u = xp_ref[...] + msg_ref[...] / dinv_ref[...]
    yt = lax.dot_general(wl_ref[...], u, (((0,), (0,)), ((), ())),
                         preferred_element_type=jnp.float32) + blc_ref[...]
    o_ref[...] = yt


def _tc1(xpt, msgt, dinv, Wl1, bl1c):
    return pl.pallas_call(
        _tc1_body,
        grid=(_NB0,),
        in_specs=[
            pl.BlockSpec((H, _RB), lambda i: (0, i)),
            pl.BlockSpec((H, _RB), lambda i: (0, i)),
            pl.BlockSpec((1, _RB), lambda i: (0, i)),
            pl.BlockSpec((H, H), lambda i: (0, 0)),
            pl.BlockSpec((H, 1), lambda i: (0, 0)),
        ],
        out_specs=pl.BlockSpec((H, _RB), lambda i: (0, i)),
        out_shape=jax.ShapeDtypeStruct((H, N0), jnp.float32),
    )(xpt, msgt, dinv, Wl1, bl1c)


def _tcf_body(xp_ref, msg_ref, dinv_ref, x1_ref, w0_ref, w1_ref,
              s0_ref, s1_ref):
    i = pl.program_id(0)
    lanes = lax.broadcasted_iota(jnp.int32, (1, _RB), 1)
    u = xp_ref[...] + msg_ref[...] / dinv_ref[...]
    y0 = lax.dot_general(w0_ref[...], u, (((0,), (0,)), ((), ())),
                         preferred_element_type=jnp.float32)
    ok0 = lanes < jnp.where(i == _NB0 - 1, _V0, _RB)
    y0 = jnp.where(ok0, y0, 0.0)

    @pl.when(i == 0)
    def _():
        s0_ref[...] = y0

    @pl.when(i != 0)
    def _():
        s0_ref[...] = s0_ref[...] + y0

    y1 = lax.dot_general(w1_ref[...], x1_ref[...], (((0,), (0,)), ((), ())),
                         preferred_element_type=jnp.float32)
    ok1 = lanes < jnp.where(i == _NB1 - 1, _V1, _RB)
    y1 = jnp.where(ok1, y1, 0.0)

    @pl.when(i == 0)
    def _():
        s1_ref[...] = y1

    @pl.when(jnp.logical_and(i != 0, i < _NB1))
    def _():
        s1_ref[...] = s1_ref[...] + y1


def _tcf(xpt, msgt, dinv, x1t, w0, w1):
    return pl.pallas_call(
        _tcf_body,
        grid=(_NB0,),
        in_specs=[
            pl.BlockSpec((H, _RB), lambda i: (0, i)),
            pl.BlockSpec((H, _RB), lambda i: (0, i)),
            pl.BlockSpec((1, _RB), lambda i: (0, i)),
            pl.BlockSpec((H, _RB), lambda i: (0, jnp.minimum(i, _NB1 - 1))),
            pl.BlockSpec((H, 1), lambda i: (0, 0)),
            pl.BlockSpec((H, 1), lambda i: (0, 0)),
        ],
        out_specs=[
            pl.BlockSpec((1, _RB), lambda i: (0, 0)),
            pl.BlockSpec((1, _RB), lambda i: (0, 0)),
        ],
        out_shape=[
            jax.ShapeDtypeStruct((1, _RB), jnp.float32),
            jax.ShapeDtypeStruct((1, _RB), jnp.float32),
        ],
    )(xpt, msgt, dinv, x1t, w0, w1)


def _tcs_body(s0_ref, s1_ref, b0_ref, b1_ref, o_ref):
    a0 = jnp.sum(s0_ref[...]) / N0 + b0_ref[0, 0]
    a1 = jnp.sum(s1_ref[...]) / N1 + b1_ref[0, 0]
    o_ref[...] = jnp.full((1, 128), a0 + a1, jnp.float32)


def _tcs(s0, s1, b0, b1):
    return pl.pallas_call(
        _tcs_body,
        out_shape=jax.ShapeDtypeStruct((1, 128), jnp.float32),
    )(s0, s1, b0, b1)


def kernel(x_0, x_1, inc_rows, inc_cols, W_in0, b_in0, W_in1, b_in1, W_l, b_l,
           W_out0, b_out0, W_out1, b_out1):
    pairs = _pack_pairs(inc_rows.astype(jnp.int32),
                        inc_cols.astype(jnp.int32))

    dinv = _tcd(_deg_kernel(pairs).reshape(NW, N0))

    x0p0 = _tc0(x_0, W_in0, b_in0.reshape(1, H), W_l[0],
                b_l[0].reshape(H, 1))
    x1s0 = _seg_v2e(x0p0.reshape(H * N0), pairs)
    msg0 = _seg_e2v(x1s0, pairs).reshape(H, N0)

    x0p1 = _tc1(x0p0, msg0, dinv, W_l[1], b_l[1].reshape(H, 1))
    x1s1 = _seg_v2e(x0p1.reshape(H * N0), pairs)
    msg1 = _seg_e2v(x1s1, pairs).reshape(H, N0)

    s0, s1 = _tcf(x0p1, msg1, dinv, x1s1.reshape(H, N1),
                  W_out0.reshape(H, 1), W_out1.reshape(H, 1))

    out = _tcs(s0, s1, b_out0.reshape(1, 1), b_out1.reshape(1, 1))
    return out[0, :1]


# speedup vs baseline: 2.7901x; 2.7901x over previous
import functools
import jax
import jax.numpy as jnp
from jax import lax
from jax.experimental import pallas as pl
from jax.experimental.pallas import tpu as pltpu
from jax.experimental.pallas import tpu_sc as plsc

N0 = 50000
N1 = 20000
NNZ = 800000
H = 64
NTILES = 16
NCORES = 2
NW = NCORES * NTILES
SHIFT = 15
MASK = (1 << SHIFT) - 1

_CHUNK = 40000
_NCH = NNZ // _CHUNK
_UNR = 10


def _make_seg(n_src, n_dst, gather_hi):
    mesh = plsc.VectorSubcoreMesh(core_axis_name="c", subcore_axis_name="s")

    @functools.partial(
        pl.kernel,
        out_type=jax.ShapeDtypeStruct((H * n_dst,), jnp.float32),
        mesh=mesh,
        compiler_params=pltpu.CompilerParams(needs_layout_passes=False),
        scratch_types=[
            pltpu.VMEM((n_src,), jnp.float32),
            pltpu.VMEM((n_dst,), jnp.float32),
            pltpu.VMEM((_CHUNK,), jnp.int32),
        ],
    )
    def seg(tab_hbm, pairs_hbm, out_hbm, tab_v, acc_v, idx_v):
        c = lax.axis_index("c")
        s = lax.axis_index("s")
        wid = c * NTILES + s

        for sweep in range(2):
            f = sweep * NW + wid
            pltpu.sync_copy(tab_hbm.at[pl.ds(pl.multiple_of(f * n_src, 8),
                                             n_src)], tab_v)

            def zero_body(i, _):
                acc_v[pl.ds(pl.multiple_of(i * 16, 8), 16)] = (
                    jnp.zeros((16,), jnp.float32))
                return 0
            lax.fori_loop(0, n_dst // 16, zero_body, 0)

            def chunk_body(a, _):
                pltpu.sync_copy(
                    pairs_hbm.at[pl.ds(pl.multiple_of(a * _CHUNK, 8),
                                       _CHUNK)], idx_v)

                @plsc.parallel_loop(0, _CHUNK // 16, unroll=_UNR)
                def _grp(g):
                    p = idx_v[pl.ds(pl.multiple_of(g * 16, 8), 16)]
                    hi = lax.shift_right_logical(p, SHIFT)
                    lo = lax.bitwise_and(p, MASK)
                    src = hi if gather_hi else lo
                    dst = lo if gather_hi else hi
                    vals = plsc.load_gather(tab_v, [src])
                    plsc.addupdate_scatter(acc_v, [dst], vals)
                return 0
            lax.fori_loop(0, _NCH, chunk_body, 0)

            pltpu.sync_copy(acc_v,
                            out_hbm.at[pl.ds(pl.multiple_of(f * n_dst, 8),
                                             n_dst)])

    return seg


_seg_v2e = _make_seg(N0, N1, True)
_seg_e2v = _make_seg(N1, N0, False)


_DPT = NNZ // NW
_DG = _DPT // 16


def _make_deg():
    mesh = plsc.VectorSubcoreMesh(core_axis_name="c", subcore_axis_name="s")

    @functools.partial(
        pl.kernel,
        out_type=jax.ShapeDtypeStruct((NW * N0,), jnp.float32),
        mesh=mesh,
        compiler_params=pltpu.CompilerParams(needs_layout_passes=False),
        scratch_types=[
            pltpu.VMEM((N0,), jnp.float32),
            pltpu.VMEM((_DPT + 16,), jnp.int32),
        ],
    )
    def deg(pairs_hbm, out_hbm, acc_v, idx_v):
        c = lax.axis_index("c")
        s = lax.axis_index("s")
        wid = c * NTILES + s

        def zero_body(i, _):
            acc_v[pl.ds(pl.multiple_of(i * 16, 8), 16)] = (
                jnp.zeros((16,), jnp.float32))
            return 0
        lax.fori_loop(0, N0 // 16, zero_body, 0)

        pltpu.sync_copy(
            pairs_hbm.at[pl.ds(pl.multiple_of(wid * _DPT, 8), _DPT)],
            idx_v.at[pl.ds(0, _DPT)])

        ones = jnp.ones((16,), jnp.float32)

        def grp_body(g, _):
            p = idx_v[pl.ds(pl.multiple_of(g * 16, 8), 16)]
            row = lax.shift_right_logical(p, SHIFT)
            plsc.addupdate_scatter(acc_v, [row], ones)
            return 0
        lax.fori_loop(0, _DG, grp_body, 0)

        p = idx_v[pl.ds(pl.multiple_of(_DG * 16, 8), 16)]
        row = lax.shift_right_logical(p, SHIFT)
        msk = lax.iota(jnp.int32, 16) < (_DPT - _DG * 16)
        plsc.addupdate_scatter(acc_v, [row], ones, mask=msk)

        pltpu.sync_copy(acc_v,
                        out_hbm.at[pl.ds(pl.multiple_of(wid * N0, 8), N0)])

    return deg


_deg_kernel = _make_deg()


_RB = 1024
_NB0 = (N0 + _RB - 1) // _RB
_NB1 = (N1 + _RB - 1) // _RB
_V0 = N0 - (_NB0 - 1) * _RB
_V1 = N1 - (_NB1 - 1) * _RB


def _pack_body(r_ref, c_ref, o_ref):
    o_ref[...] = lax.shift_left(r_ref[...], SHIFT) | c_ref[...]


def _pack_pairs(rows, cols):
    r2 = rows.reshape(NNZ // 128, 128)
    c2 = cols.reshape(NNZ // 128, 128)
    out = pl.pallas_call(
        _pack_body,
        out_shape=jax.ShapeDtypeStruct((NNZ // 128, 128), jnp.int32),
    )(r2, c2)
    return out.reshape(NNZ)


def _tcd_body(dp_ref, o_ref):
    d = jnp.sum(dp_ref[...], axis=0, keepdims=True)
    o_ref[...] = jnp.maximum(d, 1.0)


def _tcd(deg_partials):
    return pl.pallas_call(
        _tcd_body,
        grid=(_NB0,),
        in_specs=[pl.BlockSpec((NW, _RB), lambda i: (0, i))],
        out_specs=pl.BlockSpec((1, _RB), lambda i: (0, i)),
        out_shape=jax.ShapeDtypeStruct((1, N0), jnp.float32),
    )(deg_partials)


def _tc0_body(x_ref, wi_ref, bi_ref, wl_ref, blc_ref, o_ref):
    t = jnp.dot(x_ref[...], wi_ref[...],
                preferred_element_type=jnp.float32) + bi_ref[...]
    yt = lax.dot_general(wl_ref[...], t, (((0,), (1,)), ((), ())),
                         preferred_element_type=jnp.float32)
    o_ref[...] = yt + blc_ref[...]


def _tc0(x0, W_in0, b_in0, Wl0, bl0c):
    return pl.pallas_call(
        _tc0_body,
        grid=(_NB0,),
        in_specs=[
            pl.BlockSpec((_RB, 32), lambda i: (i, 0)),
            pl.BlockSpec((32, H), lambda i: (0, 0)),
            pl.BlockSpec((1, H), lambda i: (0, 0)),
            pl.BlockSpec((H, H), lambda i: (0, 0)),
            pl.BlockSpec((H, 1), lambda i: (0, 0)),
        ],
        out_specs=pl.BlockSpec((H, _RB), lambda i: (0, i)),
        out_shape=jax.ShapeDtypeStruct((H, N0), jnp.float32),
    )(x0, W_in0, b_in0, Wl0, bl0c)


def _tc1_body(xp_ref, msg_ref, dinv_ref, wl_ref, blc_ref, o_ref):
    u = xp_ref[...] + msg_ref[...] / dinv_ref[...]
    yt = lax.dot_general(wl_ref[...], u, (((0,), (0,)), ((), ())),
                         preferred_element_type=jnp.float32) + blc_ref[...]
    o_ref[...] = yt


def _tc1(xpt, msgt, dinv, Wl1, bl1c):
    return pl.pallas_call(
        _tc1_body,
        grid=(_NB0,),
        in_specs=[
            pl.BlockSpec((H, _RB), lambda i: (0, i)),
            pl.BlockSpec((H, _RB), lambda i: (0, i)),
            pl.BlockSpec((1, _RB), lambda i: (0, i)),
            pl.BlockSpec((H, H), lambda i: (0, 0)),
            pl.BlockSpec((H, 1), lambda i: (0, 0)),
        ],
        out_specs=pl.BlockSpec((H, _RB), lambda i: (0, i)),
        out_shape=jax.ShapeDtypeStruct((H, N0), jnp.float32),
    )(xpt, msgt, dinv, Wl1, bl1c)


def _tcf_body(xp_ref, msg_ref, dinv_ref, x1_ref, w0_ref, w1_ref,
              s0_ref, s1_ref):
    i = pl.program_id(0)
    lanes = lax.broadcasted_iota(jnp.int32, (1, _RB), 1)
    u = xp_ref[...] + msg_ref[...] / dinv_ref[...]
    y0 = lax.dot_general(w0_ref[...], u, (((0,), (0,)), ((), ())),
                         preferred_element_type=jnp.float32)
    ok0 = lanes < jnp.where(i == _NB0 - 1, _V0, _RB)
    y0 = jnp.where(ok0, y0, 0.0)

    @pl.when(i == 0)
    def _():
        s0_ref[...] = y0

    @pl.when(i != 0)
    def _():
        s0_ref[...] = s0_ref[...] + y0

    y1 = lax.dot_general(w1_ref[...], x1_ref[...], (((0,), (0,)), ((), ())),
                         preferred_element_type=jnp.float32)
    ok1 = lanes < jnp.where(i == _NB1 - 1, _V1, _RB)
    y1 = jnp.where(ok1, y1, 0.0)

    @pl.when(i == 0)
    def _():
        s1_ref[...] = y1

    @pl.when(jnp.logical_and(i != 0, i < _NB1))
    def _():
        s1_ref[...] = s1_ref[...] + y1


def _tcf(xpt, msgt, dinv, x1t, w0, w1):
    return pl.pallas_call(
        _tcf_body,
        grid=(_NB0,),
        in_specs=[
            pl.BlockSpec((H, _RB), lambda i: (0, i)),
            pl.BlockSpec((H, _RB), lambda i: (0, i)),
            pl.BlockSpec((1, _RB), lambda i: (0, i)),
            pl.BlockSpec((H, _RB), lambda i: (0, jnp.minimum(i, _NB1 - 1))),
            pl.BlockSpec((H, 1), lambda i: (0, 0)),
            pl.BlockSpec((H, 1), lambda i: (0, 0)),
        ],
        out_specs=[
            pl.BlockSpec((1, _RB), lambda i: (0, 0)),
            pl.BlockSpec((1, _RB), lambda i: (0, 0)),
        ],
        out_shape=[
            jax.ShapeDtypeStruct((1, _RB), jnp.float32),
            jax.ShapeDtypeStruct((1, _RB), jnp.float32),
        ],
    )(xpt, msgt, dinv, x1t, w0, w1)


def _tcs_body(s0_ref, s1_ref, b0_ref, b1_ref, o_ref):
    a0 = jnp.sum(s0_ref[...]) / N0 + b0_ref[0, 0]
    a1 = jnp.sum(s1_ref[...]) / N1 + b1_ref[0, 0]
    o_ref[...] = jnp.full((1, 128), a0 + a1, jnp.float32)


def _tcs(s0, s1, b0, b1):
    return pl.pallas_call(
        _tcs_body,
        out_shape=jax.ShapeDtypeStruct((1, 128), jnp.float32),
    )(s0, s1, b0, b1)


def kernel(x_0, x_1, inc_rows, inc_cols, W_in0, b_in0, W_in1, b_in1, W_l, b_l,
           W_out0, b_out0, W_out1, b_out1):
    pairs = _pack_pairs(inc_rows.astype(jnp.int32),
                        inc_cols.astype(jnp.int32))

    dinv = _tcd(_deg_kernel(pairs).reshape(NW, N0))

    x0p0 = _tc0(x_0, W_in0, b_in0.reshape(1, H), W_l[0],
                b_l[0].reshape(H, 1))
    x1s0 = _seg_v2e(x0p0.reshape(H * N0), pairs)
    msg0 = _seg_e2v(x1s0, pairs).reshape(H, N0)

    x0p1 = _tc1(x0p0, msg0, dinv, W_l[1], b_l[1].reshape(H, 1))
    x1s1 = _seg_v2e(x0p1.reshape(H * N0), pairs)
    msg1 = _seg_e2v(x1s1, pairs).reshape(H, N0)

    s0, s1 = _tcf(x0p1, msg1, dinv, x1s1.reshape(H, N1),
                  W_out0.reshape(H, 1), W_out1.reshape(H, 1))

    out = _tcs(s0, s1, b_out0.reshape(1, 1), b_out1.reshape(1, 1))
    return out[0, :1]


# speedup vs baseline: 2.8160x; 1.0093x over previous
import functools
import jax
import jax.numpy as jnp
from jax import lax
from jax.experimental import pallas as pl
from jax.experimental.pallas import tpu as pltpu
from jax.experimental.pallas import tpu_sc as plsc

N0 = 50000
N1 = 20000
NNZ = 800000
H = 64
NTILES = 16
NCORES = 2
NW = NCORES * NTILES
SHIFT = 15
MASK = (1 << SHIFT) - 1

_CHUNK = 40000
_NCH = NNZ // _CHUNK
_UNR = 20


def _make_seg(n_src, n_dst, gather_hi):
    mesh = plsc.VectorSubcoreMesh(core_axis_name="c", subcore_axis_name="s")

    @functools.partial(
        pl.kernel,
        out_type=jax.ShapeDtypeStruct((H * n_dst,), jnp.float32),
        mesh=mesh,
        compiler_params=pltpu.CompilerParams(needs_layout_passes=False),
        scratch_types=[
            pltpu.VMEM((n_src,), jnp.float32),
            pltpu.VMEM((n_dst,), jnp.float32),
            pltpu.VMEM((_CHUNK,), jnp.int32),
        ],
    )
    def seg(tab_hbm, pairs_hbm, out_hbm, tab_v, acc_v, idx_v):
        c = lax.axis_index("c")
        s = lax.axis_index("s")
        wid = c * NTILES + s

        for sweep in range(2):
            f = sweep * NW + wid
            pltpu.sync_copy(tab_hbm.at[pl.ds(pl.multiple_of(f * n_src, 8),
                                             n_src)], tab_v)

            def zero_body(i, _):
                acc_v[pl.ds(pl.multiple_of(i * 16, 8), 16)] = (
                    jnp.zeros((16,), jnp.float32))
                return 0
            lax.fori_loop(0, n_dst // 16, zero_body, 0)

            def chunk_body(a, _):
                pltpu.sync_copy(
                    pairs_hbm.at[pl.ds(pl.multiple_of(a * _CHUNK, 8),
                                       _CHUNK)], idx_v)

                @plsc.parallel_loop(0, _CHUNK // 16, unroll=_UNR)
                def _grp(g):
                    p = idx_v[pl.ds(pl.multiple_of(g * 16, 8), 16)]
                    hi = lax.shift_right_logical(p, SHIFT)
                    lo = lax.bitwise_and(p, MASK)
                    src = hi if gather_hi else lo
                    dst = lo if gather_hi else hi
                    vals = plsc.load_gather(tab_v, [src])
                    plsc.addupdate_scatter(acc_v, [dst], vals)
                return 0
            lax.fori_loop(0, _NCH, chunk_body, 0)

            pltpu.sync_copy(acc_v,
                            out_hbm.at[pl.ds(pl.multiple_of(f * n_dst, 8),
                                             n_dst)])

    return seg


_seg_v2e = _make_seg(N0, N1, True)
_seg_e2v = _make_seg(N1, N0, False)


_DPT = NNZ // NW
_DG = _DPT // 16


def _make_deg():
    mesh = plsc.VectorSubcoreMesh(core_axis_name="c", subcore_axis_name="s")

    @functools.partial(
        pl.kernel,
        out_type=jax.ShapeDtypeStruct((NW * N0,), jnp.float32),
        mesh=mesh,
        compiler_params=pltpu.CompilerParams(needs_layout_passes=False),
        scratch_types=[
            pltpu.VMEM((N0,), jnp.float32),
            pltpu.VMEM((_DPT + 16,), jnp.int32),
        ],
    )
    def deg(pairs_hbm, out_hbm, acc_v, idx_v):
        c = lax.axis_index("c")
        s = lax.axis_index("s")
        wid = c * NTILES + s

        def zero_body(i, _):
            acc_v[pl.ds(pl.multiple_of(i * 16, 8), 16)] = (
                jnp.zeros((16,), jnp.float32))
            return 0
        lax.fori_loop(0, N0 // 16, zero_body, 0)

        pltpu.sync_copy(
            pairs_hbm.at[pl.ds(pl.multiple_of(wid * _DPT, 8), _DPT)],
            idx_v.at[pl.ds(0, _DPT)])

        ones = jnp.ones((16,), jnp.float32)

        def grp_body(g, _):
            p = idx_v[pl.ds(pl.multiple_of(g * 16, 8), 16)]
            row = lax.shift_right_logical(p, SHIFT)
            plsc.addupdate_scatter(acc_v, [row], ones)
            return 0
        lax.fori_loop(0, _DG, grp_body, 0)

        p = idx_v[pl.ds(pl.multiple_of(_DG * 16, 8), 16)]
        row = lax.shift_right_logical(p, SHIFT)
        msk = lax.iota(jnp.int32, 16) < (_DPT - _DG * 16)
        plsc.addupdate_scatter(acc_v, [row], ones, mask=msk)

        pltpu.sync_copy(acc_v,
                        out_hbm.at[pl.ds(pl.multiple_of(wid * N0, 8), N0)])

    return deg


_deg_kernel = _make_deg()


_RB = 1024
_NB0 = (N0 + _RB - 1) // _RB
_NB1 = (N1 + _RB - 1) // _RB
_V0 = N0 - (_NB0 - 1) * _RB
_V1 = N1 - (_NB1 - 1) * _RB


def _pack_body(r_ref, c_ref, o_ref):
    o_ref[...] = lax.shift_left(r_ref[...], SHIFT) | c_ref[...]


def _pack_pairs(rows, cols):
    r2 = rows.reshape(NNZ // 128, 128)
    c2 = cols.reshape(NNZ // 128, 128)
    out = pl.pallas_call(
        _pack_body,
        out_shape=jax.ShapeDtypeStruct((NNZ // 128, 128), jnp.int32),
    )(r2, c2)
    return out.reshape(NNZ)


def _tcd_body(dp_ref, o_ref):
    d = jnp.sum(dp_ref[...], axis=0, keepdims=True)
    o_ref[...] = jnp.maximum(d, 1.0)


def _tcd(deg_partials):
    return pl.pallas_call(
        _tcd_body,
        grid=(_NB0,),
        in_specs=[pl.BlockSpec((NW, _RB), lambda i: (0, i))],
        out_specs=pl.BlockSpec((1, _RB), lambda i: (0, i)),
        out_shape=jax.ShapeDtypeStruct((1, N0), jnp.float32),
    )(deg_partials)


def _tc0_body(x_ref, wi_ref, bi_ref, wl_ref, blc_ref, o_ref):
    t = jnp.dot(x_ref[...], wi_ref[...],
                preferred_element_type=jnp.float32) + bi_ref[...]
    yt = lax.dot_general(wl_ref[...], t, (((0,), (1,)), ((), ())),
                         preferred_element_type=jnp.float32)
    o_ref[...] = yt + blc_ref[...]


def _tc0(x0, W_in0, b_in0, Wl0, bl0c):
    return pl.pallas_call(
        _tc0_body,
        grid=(_NB0,),
        in_specs=[
            pl.BlockSpec((_RB, 32), lambda i: (i, 0)),
            pl.BlockSpec((32, H), lambda i: (0, 0)),
            pl.BlockSpec((1, H), lambda i: (0, 0)),
            pl.BlockSpec((H, H), lambda i: (0, 0)),
            pl.BlockSpec((H, 1), lambda i: (0, 0)),
        ],
        out_specs=pl.BlockSpec((H, _RB), lambda i: (0, i)),
        out_shape=jax.ShapeDtypeStruct((H, N0), jnp.float32),
    )(x0, W_in0, b_in0, Wl0, bl0c)


def _tc1_body(xp_ref, msg_ref, dinv_ref, wl_ref, blc_ref, o_ref):
    u = xp_ref[...] + msg_ref[...] / dinv_ref[...]
    yt = lax.dot_general(wl_ref[...], u, (((0,), (0,)), ((), ())),
                         preferred_element_type=jnp.float32) + blc_ref[...]
    o_ref[...] = yt


def _tc1(xpt, msgt, dinv, Wl1, bl1c):
    return pl.pallas_call(
        _tc1_body,
        grid=(_NB0,),
        in_specs=[
            pl.BlockSpec((H, _RB), lambda i: (0, i)),
            pl.BlockSpec((H, _RB), lambda i: (0, i)),
            pl.BlockSpec((1, _RB), lambda i: (0, i)),
            pl.BlockSpec((H, H), lambda i: (0, 0)),
            pl.BlockSpec((H, 1), lambda i: (0, 0)),
        ],
        out_specs=pl.BlockSpec((H, _RB), lambda i: (0, i)),
        out_shape=jax.ShapeDtypeStruct((H, N0), jnp.float32),
    )(xpt, msgt, dinv, Wl1, bl1c)


def _tcf_body(xp_ref, msg_ref, dinv_ref, x1_ref, w0_ref, w1_ref,
              s0_ref, s1_ref):
    i = pl.program_id(0)
    lanes = lax.broadcasted_iota(jnp.int32, (1, _RB), 1)
    u = xp_ref[...] + msg_ref[...] / dinv_ref[...]
    y0 = lax.dot_general(w0_ref[...], u, (((0,), (0,)), ((), ())),
                         preferred_element_type=jnp.float32)
    ok0 = lanes < jnp.where(i == _NB0 - 1, _V0, _RB)
    y0 = jnp.where(ok0, y0, 0.0)

    @pl.when(i == 0)
    def _():
        s0_ref[...] = y0

    @pl.when(i != 0)
    def _():
        s0_ref[...] = s0_ref[...] + y0

    y1 = lax.dot_general(w1_ref[...], x1_ref[...], (((0,), (0,)), ((), ())),
                         preferred_element_type=jnp.float32)
    ok1 = lanes < jnp.where(i == _NB1 - 1, _V1, _RB)
    y1 = jnp.where(ok1, y1, 0.0)

    @pl.when(i == 0)
    def _():
        s1_ref[...] = y1

    @pl.when(jnp.logical_and(i != 0, i < _NB1))
    def _():
        s1_ref[...] = s1_ref[...] + y1


def _tcf(xpt, msgt, dinv, x1t, w0, w1):
    return pl.pallas_call(
        _tcf_body,
        grid=(_NB0,),
        in_specs=[
            pl.BlockSpec((H, _RB), lambda i: (0, i)),
            pl.BlockSpec((H, _RB), lambda i: (0, i)),
            pl.BlockSpec((1, _RB), lambda i: (0, i)),
            pl.BlockSpec((H, _RB), lambda i: (0, jnp.minimum(i, _NB1 - 1))),
            pl.BlockSpec((H, 1), lambda i: (0, 0)),
            pl.BlockSpec((H, 1), lambda i: (0, 0)),
        ],
        out_specs=[
            pl.BlockSpec((1, _RB), lambda i: (0, 0)),
            pl.BlockSpec((1, _RB), lambda i: (0, 0)),
        ],
        out_shape=[
            jax.ShapeDtypeStruct((1, _RB), jnp.float32),
            jax.ShapeDtypeStruct((1, _RB), jnp.float32),
        ],
    )(xpt, msgt, dinv, x1t, w0, w1)


def _tcs_body(s0_ref, s1_ref, b0_ref, b1_ref, o_ref):
    a0 = jnp.sum(s0_ref[...]) / N0 + b0_ref[0, 0]
    a1 = jnp.sum(s1_ref[...]) / N1 + b1_ref[0, 0]
    o_ref[...] = jnp.full((1, 128), a0 + a1, jnp.float32)


def _tcs(s0, s1, b0, b1):
    return pl.pallas_call(
        _tcs_body,
        out_shape=jax.ShapeDtypeStruct((1, 128), jnp.float32),
    )(s0, s1, b0, b1)


def kernel(x_0, x_1, inc_rows, inc_cols, W_in0, b_in0, W_in1, b_in1, W_l, b_l,
           W_out0, b_out0, W_out1, b_out1):
    pairs = _pack_pairs(inc_rows.astype(jnp.int32),
                        inc_cols.astype(jnp.int32))

    dinv = _tcd(_deg_kernel(pairs).reshape(NW, N0))

    x0p0 = _tc0(x_0, W_in0, b_in0.reshape(1, H), W_l[0],
                b_l[0].reshape(H, 1))
    x1s0 = _seg_v2e(x0p0.reshape(H * N0), pairs)
    msg0 = _seg_e2v(x1s0, pairs).reshape(H, N0)

    x0p1 = _tc1(x0p0, msg0, dinv, W_l[1], b_l[1].reshape(H, 1))
    x1s1 = _seg_v2e(x0p1.reshape(H * N0), pairs)
    msg1 = _seg_e2v(x1s1, pairs).reshape(H, N0)

    s0, s1 = _tcf(x0p1, msg1, dinv, x1s1.reshape(H, N1),
                  W_out0.reshape(H, 1), W_out1.reshape(H, 1))

    out = _tcs(s0, s1, b_out0.reshape(1, 1), b_out1.reshape(1, 1))
    return out[0, :1]
